# Initial kernel scaffold; baseline (speedup 1.0000x reference)
import jax
import jax.numpy as jnp
from jax.experimental import pallas as pl


def kernel(x, edge_index, batch, q_weights_0, q_weights_1, W, b):
    raise NotImplementedError("write your pallas kernel here")


# speedup vs baseline: 23.2862x; 23.2862x over previous
import functools
import numpy as np
import jax
import jax.numpy as jnp
from jax import lax
from jax.experimental import pallas as pl
from jax.experimental.pallas import tpu as pltpu
from jax.experimental.pallas import tpu_sc as plsc

N_NODES = 10000
N_EDGES = 320000
NQ = 8
DIM = 256
N_GRAPHS = 128
OUT = 2

FW = 16
N_PAD = 10240
DUMMY = N_NODES
EB = 128
NC, NS = 2, 16
NW = NC * NS
K_PER_W = 80
E_PAD = EB * K_PER_W * NW
ROWS_PER_TILE = N_PAD // NS

BLK = 512
NBLK = N_PAD // BLK

_HIGH = lax.Precision.HIGHEST


def _cnot_chain_matrix():
    i = np.arange(DIM)
    T = np.eye(DIM, dtype=np.float32)
    for q in range(NQ - 1):
        cb = (i >> (NQ - 1 - q)) & 1
        newi = np.where(cb == 1, i ^ (1 << (NQ - 2 - q)), i)
        Tq = np.zeros((DIM, DIM), np.float32)
        Tq[i, newi] = 1.0
        T = T @ Tq
    return T


_TCHAIN = _cnot_chain_matrix()
_G = (1.0 - 2.0 * ((np.arange(DIM)[:, None] >> (NQ - 1 - np.arange(NQ)[None, :])) & 1)).astype(np.float32)

def _sc_body(gather, vals_hbm, row2_hbm, col2_hbm, zeros_hbm, out_hbm,
             ridx_v, cidx_v, vals_v, stage_v, acc_sh, sem):
    cid = lax.axis_index("c")
    sid = lax.axis_index("s")
    wid = sid * NC + cid
    pltpu.sync_copy(zeros_hbm, acc_sh.at[pl.ds(sid * ROWS_PER_TILE, ROWS_PER_TILE)])
    cbase = wid * K_PER_W
    pltpu.sync_copy(col2_hbm.at[pl.ds(cbase, K_PER_W)], cidx_v)
    if gather:
        pltpu.sync_copy(row2_hbm.at[pl.ds(cbase, K_PER_W)], ridx_v)
    else:
        pltpu.sync_copy(vals_hbm, vals_v)
    plsc.subcore_barrier()

    def step(j, carry):
        if gather:
            pltpu.async_copy(vals_hbm.at[ridx_v.at[j]], vals_v, sem).wait()
        pltpu.sync_copy(vals_v, acc_sh.at[cidx_v.at[j]], add=True)
        return carry

    lax.fori_loop(0, K_PER_W, step, 0)
    plsc.subcore_barrier()
    pltpu.sync_copy(acc_sh.at[pl.ds(sid * ROWS_PER_TILE, ROWS_PER_TILE)], stage_v)
    pltpu.sync_copy(
        stage_v, out_hbm.at[pl.ds(cid * N_PAD + sid * ROWS_PER_TILE, ROWS_PER_TILE)])


@functools.lru_cache(maxsize=None)
def _make_sc(gather):
    mesh = plsc.VectorSubcoreMesh(
        core_axis_name="c", subcore_axis_name="s", num_cores=NC, num_subcores=NS)
    scratch = [
        pltpu.VMEM((K_PER_W, EB), jnp.int32),
        pltpu.VMEM((K_PER_W, EB), jnp.int32),
        pltpu.VMEM((EB, FW), jnp.float32),
        pltpu.VMEM((ROWS_PER_TILE, FW), jnp.float32),
        pltpu.VMEM_SHARED((N_PAD, FW), jnp.float32),
        pltpu.SemaphoreType.DMA,
    ]
    return pl.kernel(
        functools.partial(_sc_body, gather),
        out_type=jax.ShapeDtypeStruct((NC * N_PAD, FW), jnp.float32),
        mesh=mesh,
        scratch_types=scratch,
        compiler_params=pltpu.CompilerParams(use_tc_tiling_on_sc=False),
        name="sc_edge_scatter" if gather else "sc_degree",
    )


def _sc_scatter(*args):
    return _make_sc(True)(*args)


def _sc_degree(*args):
    return _make_sc(False)(*args)


def _prologue_body(x_ref, dega_ref, degb_ref, xs_ref, dinv_ref):
    cnt = dega_ref[:, 0:1] + degb_ref[:, 0:1]
    dinv = lax.rsqrt(cnt + 1.0)
    xs8 = x_ref[...] * dinv
    ones = jnp.ones((N_PAD, 1), jnp.float32)
    zeros = jnp.zeros((N_PAD, FW - NQ - 1), jnp.float32)
    xs_ref[...] = jnp.concatenate([xs8, ones, zeros], axis=1)
    dinv_ref[...] = jnp.broadcast_to(dinv, (N_PAD, NQ))


def _prologue(x_pad, dega, degb):
    return pl.pallas_call(
        _prologue_body,
        out_shape=(
            jax.ShapeDtypeStruct((N_PAD, FW), jnp.float32),
            jax.ShapeDtypeStruct((N_PAD, NQ), jnp.float32),
        ),
    )(x_pad, dega, degb)


def _build_t_body(w0_ref, w1_ref, tc_ref, ta_ref, tb_ref):
    jb = lax.broadcasted_iota(jnp.int32, (DIM, DIM), 0)
    ib = lax.broadcasted_iota(jnp.int32, (DIM, DIM), 1)
    for w_ref, t_ref in ((w0_ref, ta_ref), (w1_ref, tb_ref)):
        T = tc_ref[...]
        for l in range(2):
            TR = jnp.ones((DIM, DIM), jnp.float32)
            for q in range(NQ):
                ang = w_ref[l, q] * 0.5
                c = jnp.cos(ang)
                s = jnp.sin(ang)
                jbq = (jb >> (NQ - 1 - q)) & 1
                ibq = (ib >> (NQ - 1 - q)) & 1
                TR = TR * jnp.where(jbq == ibq, c, jnp.where(jbq > ibq, -s, s))
            T = jnp.dot(T, TR, precision=_HIGH, preferred_element_type=jnp.float32)
            if l == 0:
                T = jnp.dot(T, tc_ref[...], precision=_HIGH,
                            preferred_element_type=jnp.float32)
        t_ref[...] = T


def _build_t(w0, w1, tchain):
    return pl.pallas_call(
        _build_t_body,
        in_specs=[
            pl.BlockSpec(memory_space=pltpu.SMEM),
            pl.BlockSpec(memory_space=pltpu.SMEM),
            pl.BlockSpec((DIM, DIM), lambda: (0, 0)),
        ],
        out_shape=(
            jax.ShapeDtypeStruct((DIM, DIM), jnp.float32),
            jax.ShapeDtypeStruct((DIM, DIM), jnp.float32),
        ),
    )(w0, w1, tchain)


def _dense_body(first_layer, acca_ref, accb_ref, xs_ref, dinv_ref, t_ref, g_ref, out_ref):
    dinv8 = dinv_ref[...]
    feats = dinv8 * (acca_ref[:, :NQ] + accb_ref[:, :NQ] + xs_ref[:, :NQ])
    phi = feats * 0.5 + (np.pi / 4.0)
    cv = jnp.cos(phi)
    sv = jnp.sin(phi)
    ib = lax.broadcasted_iota(jnp.int32, (1, DIM), 1)
    S = jnp.ones((BLK, DIM), jnp.float32)
    for q in range(NQ):
        bit = (ib >> (NQ - 1 - q)) & 1
        S = S * jnp.where(bit == 1, sv[:, q:q + 1], cv[:, q:q + 1])
    F = jnp.dot(S, t_ref[...], precision=_HIGH, preferred_element_type=jnp.float32)
    P = F * F
    z = jnp.dot(P, g_ref[...], precision=_HIGH, preferred_element_type=jnp.float32)
    h = jnp.where(z >= 0.0, z, 0.2 * z)
    if first_layer:
        ones = jnp.ones((BLK, 1), jnp.float32)
        zeros = jnp.zeros((BLK, FW - NQ - 1), jnp.float32)
        out_ref[...] = jnp.concatenate([dinv8 * h, ones, zeros], axis=1)
    else:
        out_ref[...] = h


def _dense(first_layer, acca, accb, xs16, dinv8, T, G):
    width = FW if first_layer else NQ
    return pl.pallas_call(
        functools.partial(_dense_body, first_layer),
        grid=(NBLK,),
        in_specs=[
            pl.BlockSpec((BLK, FW), lambda i: (i, 0)),
            pl.BlockSpec((BLK, FW), lambda i: (i, 0)),
            pl.BlockSpec((BLK, FW), lambda i: (i, 0)),
            pl.BlockSpec((BLK, NQ), lambda i: (i, 0)),
            pl.BlockSpec((DIM, DIM), lambda i: (0, 0)),
            pl.BlockSpec((DIM, NQ), lambda i: (0, 0)),
        ],
        out_specs=pl.BlockSpec((BLK, width), lambda i: (i, 0)),
        out_shape=jax.ShapeDtypeStruct((N_PAD, width), jnp.float32),
    )(acca, accb, xs16, dinv8, T, G)


def _pool_body(h_ref, batch_ref, w_ref, b_ref, out_ref, sums, cnts):
    i = pl.program_id(0)

    @pl.when(i == 0)
    def _():
        sums[...] = jnp.zeros((N_GRAPHS, NQ), jnp.float32)
        cnts[...] = jnp.zeros((N_GRAPHS, 1), jnp.float32)

    onehot = (batch_ref[...] == lax.broadcasted_iota(jnp.int32, (1, N_GRAPHS), 1)
              ).astype(jnp.float32)
    sums[...] += lax.dot_general(onehot, h_ref[...], (((0,), (0,)), ((), ())),
                                 precision=_HIGH, preferred_element_type=jnp.float32)
    cnts[...] += lax.dot_general(onehot, jnp.ones((BLK, 1), jnp.float32),
                                 (((0,), (0,)), ((), ())),
                                 precision=_HIGH, preferred_element_type=jnp.float32)

    @pl.when(i == NBLK - 1)
    def _():
        pooled = sums[...] / jnp.maximum(cnts[...], 1.0)
        out_ref[...] = lax.dot_general(pooled, w_ref[...], (((1,), (1,)), ((), ())),
                                       precision=_HIGH,
                                       preferred_element_type=jnp.float32) + b_ref[...]


def _pool(h2, batch2, W, b2):
    return pl.pallas_call(
        _pool_body,
        grid=(NBLK,),
        in_specs=[
            pl.BlockSpec((BLK, NQ), lambda i: (i, 0)),
            pl.BlockSpec((BLK, 1), lambda i: (i, 0)),
            pl.BlockSpec((OUT, NQ), lambda i: (0, 0)),
            pl.BlockSpec((1, OUT), lambda i: (0, 0)),
        ],
        out_specs=pl.BlockSpec((N_GRAPHS, OUT), lambda i: (0, 0)),
        out_shape=jax.ShapeDtypeStruct((N_GRAPHS, OUT), jnp.float32),
        scratch_shapes=[
            pltpu.VMEM((N_GRAPHS, NQ), jnp.float32),
            pltpu.VMEM((N_GRAPHS, 1), jnp.float32),
        ],
    )(h2, batch2, W, b2)


def kernel(x, edge_index, batch, q_weights_0, q_weights_1, W, b):
    row = edge_index[0].astype(jnp.int32)
    col = edge_index[1].astype(jnp.int32)
    pad = jnp.full((E_PAD - N_EDGES,), DUMMY, jnp.int32)
    row2 = jnp.concatenate([row, pad]).reshape(NW * K_PER_W, EB)
    col2 = jnp.concatenate([col, pad]).reshape(NW * K_PER_W, EB)
    zeros_tile = jnp.zeros((ROWS_PER_TILE, FW), jnp.float32)
    ones_eb = jnp.ones((EB, FW), jnp.float32)
    x_pad = jnp.concatenate(
        [x, jnp.zeros((N_PAD - N_NODES, NQ), jnp.float32)], axis=0)
    batch2 = jnp.concatenate(
        [batch.astype(jnp.int32), jnp.full((N_PAD - N_NODES,), -1, jnp.int32)]
    ).reshape(N_PAD, 1)

    degacc = _sc_degree(ones_eb, row2, col2, zeros_tile)
    xs16, dinv8 = _prologue(x_pad, degacc[:N_PAD], degacc[N_PAD:])
    Ta, Tb = _build_t(q_weights_0, q_weights_1, jnp.asarray(_TCHAIN))

    acc1 = _sc_scatter(xs16, row2, col2, zeros_tile)
    xs16_2 = _dense(True, acc1[:N_PAD], acc1[N_PAD:], xs16, dinv8, Ta,
                    jnp.asarray(_G))
    acc2 = _sc_scatter(xs16_2, row2, col2, zeros_tile)
    h2 = _dense(False, acc2[:N_PAD], acc2[N_PAD:], xs16_2, dinv8, Tb,
                jnp.asarray(_G))
    return _pool(h2, batch2, W, b.reshape(1, OUT))


# speedup vs baseline: 27.5749x; 1.1842x over previous
import functools
import numpy as np
import jax
import jax.numpy as jnp
from jax import lax
from jax.experimental import pallas as pl
from jax.experimental.pallas import tpu as pltpu
from jax.experimental.pallas import tpu_sc as plsc

N_NODES = 10000
N_EDGES = 320000
NQ = 8
DIM = 256
N_GRAPHS = 128
OUT = 2

FW = 16
N_PAD = 10240
DUMMY = N_NODES
EB = 128
NC, NS = 2, 16
NW = NC * NS
K_PER_W = 80
E_PAD = EB * K_PER_W * NW
ROWS_PER_TILE = N_PAD // NS

BLK = 512
NBLK = N_PAD // BLK

_HIGH = lax.Precision.HIGHEST


def _cnot_chain_matrix():
    i = np.arange(DIM)
    T = np.eye(DIM, dtype=np.float32)
    for q in range(NQ - 1):
        cb = (i >> (NQ - 1 - q)) & 1
        newi = np.where(cb == 1, i ^ (1 << (NQ - 2 - q)), i)
        Tq = np.zeros((DIM, DIM), np.float32)
        Tq[i, newi] = 1.0
        T = T @ Tq
    return T


_TCHAIN = _cnot_chain_matrix()
_G = (1.0 - 2.0 * ((np.arange(DIM)[:, None] >> (NQ - 1 - np.arange(NQ)[None, :])) & 1)).astype(np.float32)

NBUF = 8


def _sc_body(gather, vals_hbm, row2_hbm, col2_hbm, zeros_hbm, out_hbm,
             ridx_v, cidx_v, vals_v, stage_v, acc_sh, *sems):
    cid = lax.axis_index("c")
    sid = lax.axis_index("s")
    wid = sid * NC + cid
    pltpu.sync_copy(zeros_hbm, acc_sh.at[pl.ds(sid * ROWS_PER_TILE, ROWS_PER_TILE)])
    cbase = wid * K_PER_W
    pltpu.sync_copy(col2_hbm.at[pl.ds(cbase, K_PER_W)], cidx_v)
    if gather:
        pltpu.sync_copy(row2_hbm.at[pl.ds(cbase, K_PER_W)], ridx_v)
    else:
        pltpu.sync_copy(vals_hbm, vals_v.at[0])
    plsc.subcore_barrier()

    if gather:
        for b in range(NBUF):
            pltpu.async_copy(vals_hbm.at[ridx_v.at[b]], vals_v.at[b], sems[b])

        def step(m, carry):
            for b in range(NBUF):
                j = m * NBUF + b
                pltpu.make_async_copy(
                    vals_hbm.at[ridx_v.at[j]], vals_v.at[b], sems[b]).wait()
                pltpu.sync_copy(vals_v.at[b], acc_sh.at[cidx_v.at[j]], add=True)

                @pl.when(j + NBUF < K_PER_W)
                def _():
                    pltpu.async_copy(
                        vals_hbm.at[ridx_v.at[j + NBUF]], vals_v.at[b], sems[b])
            return carry

        lax.fori_loop(0, K_PER_W // NBUF, step, 0)
    else:
        def step(j, carry):
            pltpu.sync_copy(vals_v.at[0], acc_sh.at[cidx_v.at[j]], add=True)
            return carry

        lax.fori_loop(0, K_PER_W, step, 0)
    plsc.subcore_barrier()
    pltpu.sync_copy(acc_sh.at[pl.ds(sid * ROWS_PER_TILE, ROWS_PER_TILE)], stage_v)
    pltpu.sync_copy(
        stage_v, out_hbm.at[pl.ds(cid * N_PAD + sid * ROWS_PER_TILE, ROWS_PER_TILE)])


@functools.lru_cache(maxsize=None)
def _make_sc(gather):
    mesh = plsc.VectorSubcoreMesh(
        core_axis_name="c", subcore_axis_name="s", num_cores=NC, num_subcores=NS)
    scratch = [
        pltpu.VMEM((K_PER_W, EB), jnp.int32),
        pltpu.VMEM((K_PER_W, EB), jnp.int32),
        pltpu.VMEM((NBUF, EB, FW), jnp.float32),
        pltpu.VMEM((ROWS_PER_TILE, FW), jnp.float32),
        pltpu.VMEM_SHARED((N_PAD, FW), jnp.float32),
    ] + [pltpu.SemaphoreType.DMA] * NBUF
    return pl.kernel(
        functools.partial(_sc_body, gather),
        out_type=jax.ShapeDtypeStruct((NC * N_PAD, FW), jnp.float32),
        mesh=mesh,
        scratch_types=scratch,
        compiler_params=pltpu.CompilerParams(use_tc_tiling_on_sc=False),
        name="sc_edge_scatter" if gather else "sc_degree",
    )


def _sc_scatter(*args):
    return _make_sc(True)(*args)


def _sc_degree(*args):
    return _make_sc(False)(*args)


def _prologue_body(x_ref, dega_ref, degb_ref, xs_ref, dinv_ref):
    cnt = dega_ref[:, 0:1] + degb_ref[:, 0:1]
    dinv = lax.rsqrt(cnt + 1.0)
    xs8 = x_ref[...] * dinv
    ones = jnp.ones((N_PAD, 1), jnp.float32)
    zeros = jnp.zeros((N_PAD, FW - NQ - 1), jnp.float32)
    xs_ref[...] = jnp.concatenate([xs8, ones, zeros], axis=1)
    dinv_ref[...] = jnp.broadcast_to(dinv, (N_PAD, NQ))


def _prologue(x_pad, dega, degb):
    return pl.pallas_call(
        _prologue_body,
        out_shape=(
            jax.ShapeDtypeStruct((N_PAD, FW), jnp.float32),
            jax.ShapeDtypeStruct((N_PAD, NQ), jnp.float32),
        ),
    )(x_pad, dega, degb)


def _build_t_body(w0_ref, w1_ref, tc_ref, ta_ref, tb_ref):
    jb = lax.broadcasted_iota(jnp.int32, (DIM, DIM), 0)
    ib = lax.broadcasted_iota(jnp.int32, (DIM, DIM), 1)
    for w_ref, t_ref in ((w0_ref, ta_ref), (w1_ref, tb_ref)):
        T = tc_ref[...]
        for l in range(2):
            TR = jnp.ones((DIM, DIM), jnp.float32)
            for q in range(NQ):
                ang = w_ref[l, q] * 0.5
                c = jnp.cos(ang)
                s = jnp.sin(ang)
                jbq = (jb >> (NQ - 1 - q)) & 1
                ibq = (ib >> (NQ - 1 - q)) & 1
                TR = TR * jnp.where(jbq == ibq, c, jnp.where(jbq > ibq, -s, s))
            T = jnp.dot(T, TR, precision=_HIGH, preferred_element_type=jnp.float32)
            if l == 0:
                T = jnp.dot(T, tc_ref[...], precision=_HIGH,
                            preferred_element_type=jnp.float32)
        t_ref[...] = T


def _build_t(w0, w1, tchain):
    return pl.pallas_call(
        _build_t_body,
        in_specs=[
            pl.BlockSpec(memory_space=pltpu.SMEM),
            pl.BlockSpec(memory_space=pltpu.SMEM),
            pl.BlockSpec((DIM, DIM), lambda: (0, 0)),
        ],
        out_shape=(
            jax.ShapeDtypeStruct((DIM, DIM), jnp.float32),
            jax.ShapeDtypeStruct((DIM, DIM), jnp.float32),
        ),
    )(w0, w1, tchain)


def _dense_body(first_layer, acca_ref, accb_ref, xs_ref, dinv_ref, t_ref, g_ref, out_ref):
    dinv8 = dinv_ref[...]
    feats = dinv8 * (acca_ref[:, :NQ] + accb_ref[:, :NQ] + xs_ref[:, :NQ])
    phi = feats * 0.5 + (np.pi / 4.0)
    cv = jnp.cos(phi)
    sv = jnp.sin(phi)
    ib = lax.broadcasted_iota(jnp.int32, (1, DIM), 1)
    S = jnp.ones((BLK, DIM), jnp.float32)
    for q in range(NQ):
        bit = (ib >> (NQ - 1 - q)) & 1
        S = S * jnp.where(bit == 1, sv[:, q:q + 1], cv[:, q:q + 1])
    F = jnp.dot(S, t_ref[...], precision=_HIGH, preferred_element_type=jnp.float32)
    P = F * F
    z = jnp.dot(P, g_ref[...], precision=_HIGH, preferred_element_type=jnp.float32)
    h = jnp.where(z >= 0.0, z, 0.2 * z)
    if first_layer:
        ones = jnp.ones((BLK, 1), jnp.float32)
        zeros = jnp.zeros((BLK, FW - NQ - 1), jnp.float32)
        out_ref[...] = jnp.concatenate([dinv8 * h, ones, zeros], axis=1)
    else:
        out_ref[...] = h


def _dense(first_layer, acca, accb, xs16, dinv8, T, G):
    width = FW if first_layer else NQ
    return pl.pallas_call(
        functools.partial(_dense_body, first_layer),
        grid=(NBLK,),
        in_specs=[
            pl.BlockSpec((BLK, FW), lambda i: (i, 0)),
            pl.BlockSpec((BLK, FW), lambda i: (i, 0)),
            pl.BlockSpec((BLK, FW), lambda i: (i, 0)),
            pl.BlockSpec((BLK, NQ), lambda i: (i, 0)),
            pl.BlockSpec((DIM, DIM), lambda i: (0, 0)),
            pl.BlockSpec((DIM, NQ), lambda i: (0, 0)),
        ],
        out_specs=pl.BlockSpec((BLK, width), lambda i: (i, 0)),
        out_shape=jax.ShapeDtypeStruct((N_PAD, width), jnp.float32),
    )(acca, accb, xs16, dinv8, T, G)


def _pool_body(h_ref, batch_ref, w_ref, b_ref, out_ref, sums, cnts):
    i = pl.program_id(0)

    @pl.when(i == 0)
    def _():
        sums[...] = jnp.zeros((N_GRAPHS, NQ), jnp.float32)
        cnts[...] = jnp.zeros((N_GRAPHS, 1), jnp.float32)

    onehot = (batch_ref[...] == lax.broadcasted_iota(jnp.int32, (1, N_GRAPHS), 1)
              ).astype(jnp.float32)
    sums[...] += lax.dot_general(onehot, h_ref[...], (((0,), (0,)), ((), ())),
                                 precision=_HIGH, preferred_element_type=jnp.float32)
    cnts[...] += lax.dot_general(onehot, jnp.ones((BLK, 1), jnp.float32),
                                 (((0,), (0,)), ((), ())),
                                 precision=_HIGH, preferred_element_type=jnp.float32)

    @pl.when(i == NBLK - 1)
    def _():
        pooled = sums[...] / jnp.maximum(cnts[...], 1.0)
        out_ref[...] = lax.dot_general(pooled, w_ref[...], (((1,), (1,)), ((), ())),
                                       precision=_HIGH,
                                       preferred_element_type=jnp.float32) + b_ref[...]


def _pool(h2, batch2, W, b2):
    return pl.pallas_call(
        _pool_body,
        grid=(NBLK,),
        in_specs=[
            pl.BlockSpec((BLK, NQ), lambda i: (i, 0)),
            pl.BlockSpec((BLK, 1), lambda i: (i, 0)),
            pl.BlockSpec((OUT, NQ), lambda i: (0, 0)),
            pl.BlockSpec((1, OUT), lambda i: (0, 0)),
        ],
        out_specs=pl.BlockSpec((N_GRAPHS, OUT), lambda i: (0, 0)),
        out_shape=jax.ShapeDtypeStruct((N_GRAPHS, OUT), jnp.float32),
        scratch_shapes=[
            pltpu.VMEM((N_GRAPHS, NQ), jnp.float32),
            pltpu.VMEM((N_GRAPHS, 1), jnp.float32),
        ],
    )(h2, batch2, W, b2)


def kernel(x, edge_index, batch, q_weights_0, q_weights_1, W, b):
    row = edge_index[0].astype(jnp.int32)
    col = edge_index[1].astype(jnp.int32)
    pad = jnp.full((E_PAD - N_EDGES,), DUMMY, jnp.int32)
    row2 = jnp.concatenate([row, pad]).reshape(NW * K_PER_W, EB)
    col2 = jnp.concatenate([col, pad]).reshape(NW * K_PER_W, EB)
    zeros_tile = jnp.zeros((ROWS_PER_TILE, FW), jnp.float32)
    ones_eb = jnp.ones((EB, FW), jnp.float32)
    x_pad = jnp.concatenate(
        [x, jnp.zeros((N_PAD - N_NODES, NQ), jnp.float32)], axis=0)
    batch2 = jnp.concatenate(
        [batch.astype(jnp.int32), jnp.full((N_PAD - N_NODES,), -1, jnp.int32)]
    ).reshape(N_PAD, 1)

    degacc = _sc_degree(ones_eb, row2, col2, zeros_tile)
    xs16, dinv8 = _prologue(x_pad, degacc[:N_PAD], degacc[N_PAD:])
    Ta, Tb = _build_t(q_weights_0, q_weights_1, jnp.asarray(_TCHAIN))

    acc1 = _sc_scatter(xs16, row2, col2, zeros_tile)
    xs16_2 = _dense(True, acc1[:N_PAD], acc1[N_PAD:], xs16, dinv8, Ta,
                    jnp.asarray(_G))
    acc2 = _sc_scatter(xs16_2, row2, col2, zeros_tile)
    h2 = _dense(False, acc2[:N_PAD], acc2[N_PAD:], xs16_2, dinv8, Tb,
                jnp.asarray(_G))
    return _pool(h2, batch2, W, b.reshape(1, OUT))


# speedup vs baseline: 32.3234x; 1.1722x over previous
import functools
import numpy as np
import jax
import jax.numpy as jnp
from jax import lax
from jax.experimental import pallas as pl
from jax.experimental.pallas import tpu as pltpu
from jax.experimental.pallas import tpu_sc as plsc

N_NODES = 10000
N_EDGES = 320000
NQ = 8
DIM = 256
N_GRAPHS = 128
OUT = 2

FW = 16
N_PAD = 10240
DUMMY = N_NODES
EB = 128
NC, NS = 2, 16
NW = NC * NS
K_PER_W = 80
E_PAD = EB * K_PER_W * NW
ROWS_PER_TILE = N_PAD // NS

BLK = 1024
NBLK = N_PAD // BLK

_HIGH = lax.Precision.HIGHEST


def _split_bf16(a):
    hi = a.astype(jnp.bfloat16)
    lo = (a - hi.astype(jnp.float32)).astype(jnp.bfloat16)
    return hi, lo


def _mm(a, b, dims=(((1,), (0,)), ((), ()))):
    return lax.dot_general(a, b, dims, preferred_element_type=jnp.float32)


def _dot3(a, b):
    ah, al = _split_bf16(a)
    bh, bl = _split_bf16(b)
    return _mm(ah, bh) + (_mm(al, bh) + _mm(ah, bl))


def _dot2(a, b_exact, dims=(((1,), (0,)), ((), ()))):
    ah, al = _split_bf16(a)
    bh = b_exact.astype(jnp.bfloat16)
    return _mm(ah, bh, dims) + _mm(al, bh, dims)


def _cnot_chain_matrix():
    i = np.arange(DIM)
    T = np.eye(DIM, dtype=np.float32)
    for q in range(NQ - 1):
        cb = (i >> (NQ - 1 - q)) & 1
        newi = np.where(cb == 1, i ^ (1 << (NQ - 2 - q)), i)
        Tq = np.zeros((DIM, DIM), np.float32)
        Tq[i, newi] = 1.0
        T = T @ Tq
    return T


_TCHAIN = _cnot_chain_matrix()
_G = (1.0 - 2.0 * ((np.arange(DIM)[:, None] >> (NQ - 1 - np.arange(NQ)[None, :])) & 1)).astype(np.float32)

NBUF = 16


def _sc_body(gather, vals_hbm, row2_hbm, col2_hbm, zeros_hbm, out_hbm,
             ridx_v, cidx_v, vals_v, stage_v, acc_sh, *sems):
    cid = lax.axis_index("c")
    sid = lax.axis_index("s")
    wid = sid * NC + cid
    pltpu.sync_copy(zeros_hbm, acc_sh.at[pl.ds(sid * ROWS_PER_TILE, ROWS_PER_TILE)])
    cbase = wid * K_PER_W
    pltpu.sync_copy(col2_hbm.at[pl.ds(cbase, K_PER_W)], cidx_v)
    if gather:
        pltpu.sync_copy(row2_hbm.at[pl.ds(cbase, K_PER_W)], ridx_v)
    else:
        pltpu.sync_copy(vals_hbm, vals_v.at[0])
    plsc.subcore_barrier()

    if gather:
        for b in range(NBUF):
            pltpu.async_copy(vals_hbm.at[ridx_v.at[b]], vals_v.at[b], sems[b])

        def step(m, carry):
            for b in range(NBUF):
                j = m * NBUF + b
                pltpu.make_async_copy(
                    vals_hbm.at[ridx_v.at[j]], vals_v.at[b], sems[b]).wait()
                pltpu.sync_copy(vals_v.at[b], acc_sh.at[cidx_v.at[j]], add=True)

                @pl.when(j + NBUF < K_PER_W)
                def _():
                    pltpu.async_copy(
                        vals_hbm.at[ridx_v.at[j + NBUF]], vals_v.at[b], sems[b])
            return carry

        lax.fori_loop(0, K_PER_W // NBUF, step, 0)
    else:
        def step(j, carry):
            pltpu.sync_copy(vals_v.at[0], acc_sh.at[cidx_v.at[j]], add=True)
            return carry

        lax.fori_loop(0, K_PER_W, step, 0)
    plsc.subcore_barrier()
    pltpu.sync_copy(acc_sh.at[pl.ds(sid * ROWS_PER_TILE, ROWS_PER_TILE)], stage_v)
    pltpu.sync_copy(
        stage_v, out_hbm.at[pl.ds(cid * N_PAD + sid * ROWS_PER_TILE, ROWS_PER_TILE)])


@functools.lru_cache(maxsize=None)
def _make_sc(gather):
    mesh = plsc.VectorSubcoreMesh(
        core_axis_name="c", subcore_axis_name="s", num_cores=NC, num_subcores=NS)
    scratch = [
        pltpu.VMEM((K_PER_W, EB), jnp.int32),
        pltpu.VMEM((K_PER_W, EB), jnp.int32),
        pltpu.VMEM((NBUF, EB, FW), jnp.float32),
        pltpu.VMEM((ROWS_PER_TILE, FW), jnp.float32),
        pltpu.VMEM_SHARED((N_PAD, FW), jnp.float32),
    ] + [pltpu.SemaphoreType.DMA] * NBUF
    return pl.kernel(
        functools.partial(_sc_body, gather),
        out_type=jax.ShapeDtypeStruct((NC * N_PAD, FW), jnp.float32),
        mesh=mesh,
        scratch_types=scratch,
        compiler_params=pltpu.CompilerParams(use_tc_tiling_on_sc=False),
        name="sc_edge_scatter" if gather else "sc_degree",
    )


def _sc_scatter(*args):
    return _make_sc(True)(*args)


def _sc_degree(*args):
    return _make_sc(False)(*args)


def _prologue_body(x_ref, deg_ref, xs_ref, dinv_ref):
    cnt = deg_ref[:N_PAD, 0:1] + deg_ref[N_PAD:, 0:1]
    dinv = lax.rsqrt(cnt + 1.0)
    xs8 = x_ref[...] * dinv
    ones = jnp.ones((N_PAD, 1), jnp.float32)
    zeros = jnp.zeros((N_PAD, FW - NQ - 1), jnp.float32)
    xs_ref[...] = jnp.concatenate([xs8, ones, zeros], axis=1)
    dinv_ref[...] = jnp.broadcast_to(dinv, (N_PAD, NQ))


def _prologue(x_pad, degacc):
    return pl.pallas_call(
        _prologue_body,
        out_shape=(
            jax.ShapeDtypeStruct((N_PAD, FW), jnp.float32),
            jax.ShapeDtypeStruct((N_PAD, NQ), jnp.float32),
        ),
    )(x_pad, degacc)


def _build_t_body(w0_ref, w1_ref, tc_ref, ta_ref, tb_ref):
    jb = lax.broadcasted_iota(jnp.int32, (DIM, DIM), 0)
    ib = lax.broadcasted_iota(jnp.int32, (DIM, DIM), 1)
    for w_ref, t_ref in ((w0_ref, ta_ref), (w1_ref, tb_ref)):
        T = tc_ref[...]
        for l in range(2):
            TR = jnp.ones((DIM, DIM), jnp.float32)
            for q in range(NQ):
                ang = w_ref[l, q] * 0.5
                c = jnp.cos(ang)
                s = jnp.sin(ang)
                jbq = (jb >> (NQ - 1 - q)) & 1
                ibq = (ib >> (NQ - 1 - q)) & 1
                TR = TR * jnp.where(jbq == ibq, c, jnp.where(jbq > ibq, -s, s))
            T = jnp.dot(T, TR, precision=_HIGH, preferred_element_type=jnp.float32)
            if l == 0:
                T = jnp.dot(T, tc_ref[...], precision=_HIGH,
                            preferred_element_type=jnp.float32)
        t_ref[...] = T


def _build_t(w0, w1, tchain):
    return pl.pallas_call(
        _build_t_body,
        in_specs=[
            pl.BlockSpec(memory_space=pltpu.SMEM),
            pl.BlockSpec(memory_space=pltpu.SMEM),
            pl.BlockSpec((DIM, DIM), lambda: (0, 0)),
        ],
        out_shape=(
            jax.ShapeDtypeStruct((DIM, DIM), jnp.float32),
            jax.ShapeDtypeStruct((DIM, DIM), jnp.float32),
        ),
    )(w0, w1, tchain)


def _dense_body(first_layer, acca_ref, accb_ref, xs_ref, dinv_ref, t_ref, g_ref, out_ref):
    dinv8 = dinv_ref[...]
    feats = dinv8 * (acca_ref[:, :NQ] + accb_ref[:, :NQ] + xs_ref[:, :NQ])
    phi = feats * 0.5 + (np.pi / 4.0)
    cv = jnp.cos(phi)
    sv = jnp.sin(phi)
    ib = lax.broadcasted_iota(jnp.int32, (1, DIM), 1)
    S = jnp.ones((BLK, DIM), jnp.float32)
    for q in range(NQ):
        bit = (ib >> (NQ - 1 - q)) & 1
        S = S * jnp.where(bit == 1, sv[:, q:q + 1], cv[:, q:q + 1])
    F = _dot3(S, t_ref[...])
    P = F * F
    z = _dot2(P, g_ref[...])
    h = jnp.where(z >= 0.0, z, 0.2 * z)
    if first_layer:
        ones = jnp.ones((BLK, 1), jnp.float32)
        zeros = jnp.zeros((BLK, FW - NQ - 1), jnp.float32)
        out_ref[...] = jnp.concatenate([dinv8 * h, ones, zeros], axis=1)
    else:
        out_ref[...] = h


def _dense(first_layer, acc, xs16, dinv8, T, G):
    width = FW if first_layer else NQ
    return pl.pallas_call(
        functools.partial(_dense_body, first_layer),
        grid=(NBLK,),
        in_specs=[
            pl.BlockSpec((BLK, FW), lambda i: (i, 0)),
            pl.BlockSpec((BLK, FW), lambda i: (i + NBLK, 0)),
            pl.BlockSpec((BLK, FW), lambda i: (i, 0)),
            pl.BlockSpec((BLK, NQ), lambda i: (i, 0)),
            pl.BlockSpec((DIM, DIM), lambda i: (0, 0)),
            pl.BlockSpec((DIM, NQ), lambda i: (0, 0)),
        ],
        out_specs=pl.BlockSpec((BLK, width), lambda i: (i, 0)),
        out_shape=jax.ShapeDtypeStruct((N_PAD, width), jnp.float32),
    )(acc, acc, xs16, dinv8, T, G)


def _pool_body(h_ref, batch_ref, w_ref, b_ref, out_ref, sums, cnts):
    i = pl.program_id(0)

    @pl.when(i == 0)
    def _():
        sums[...] = jnp.zeros((N_GRAPHS, NQ), jnp.float32)
        cnts[...] = jnp.zeros((N_GRAPHS, 1), jnp.float32)

    onehot = (batch_ref[...] == lax.broadcasted_iota(jnp.int32, (1, N_GRAPHS), 1)
              ).astype(jnp.bfloat16)
    hh, hl = _split_bf16(h_ref[...])
    tdims = (((0,), (0,)), ((), ()))
    sums[...] += _mm(onehot, hh, tdims) + _mm(onehot, hl, tdims)
    cnts[...] += _mm(onehot, jnp.ones((BLK, 1), jnp.bfloat16), tdims)

    @pl.when(i == NBLK - 1)
    def _():
        pooled = sums[...] / jnp.maximum(cnts[...], 1.0)
        out_ref[...] = lax.dot_general(pooled, w_ref[...], (((1,), (1,)), ((), ())),
                                       precision=_HIGH,
                                       preferred_element_type=jnp.float32) + b_ref[...]


def _pool(h2, batch2, W, b2):
    return pl.pallas_call(
        _pool_body,
        grid=(NBLK,),
        in_specs=[
            pl.BlockSpec((BLK, NQ), lambda i: (i, 0)),
            pl.BlockSpec((BLK, 1), lambda i: (i, 0)),
            pl.BlockSpec((OUT, NQ), lambda i: (0, 0)),
            pl.BlockSpec((1, OUT), lambda i: (0, 0)),
        ],
        out_specs=pl.BlockSpec((N_GRAPHS, OUT), lambda i: (0, 0)),
        out_shape=jax.ShapeDtypeStruct((N_GRAPHS, OUT), jnp.float32),
        scratch_shapes=[
            pltpu.VMEM((N_GRAPHS, NQ), jnp.float32),
            pltpu.VMEM((N_GRAPHS, 1), jnp.float32),
        ],
    )(h2, batch2, W, b2)


def kernel(x, edge_index, batch, q_weights_0, q_weights_1, W, b):
    row = edge_index[0].astype(jnp.int32)
    col = edge_index[1].astype(jnp.int32)
    pad = jnp.full((E_PAD - N_EDGES,), DUMMY, jnp.int32)
    row2 = jnp.concatenate([row, pad]).reshape(NW * K_PER_W, EB)
    col2 = jnp.concatenate([col, pad]).reshape(NW * K_PER_W, EB)
    zeros_tile = jnp.zeros((ROWS_PER_TILE, FW), jnp.float32)
    ones_eb = jnp.ones((EB, FW), jnp.float32)
    x_pad = jnp.concatenate(
        [x, jnp.zeros((N_PAD - N_NODES, NQ), jnp.float32)], axis=0)
    batch2 = jnp.concatenate(
        [batch.astype(jnp.int32), jnp.full((N_PAD - N_NODES,), -1, jnp.int32)]
    ).reshape(N_PAD, 1)

    degacc = _sc_degree(ones_eb, row2, col2, zeros_tile)
    xs16, dinv8 = _prologue(x_pad, degacc)
    Ta, Tb = _build_t(q_weights_0, q_weights_1, jnp.asarray(_TCHAIN))

    acc1 = _sc_scatter(xs16, row2, col2, zeros_tile)
    xs16_2 = _dense(True, acc1, xs16, dinv8, Ta, jnp.asarray(_G))
    acc2 = _sc_scatter(xs16_2, row2, col2, zeros_tile)
    h2 = _dense(False, acc2, xs16_2, dinv8, Tb, jnp.asarray(_G))
    return _pool(h2, batch2, W, b.reshape(1, OUT))


# speedup vs baseline: 33.3784x; 1.0326x over previous
import functools
import numpy as np
import jax
import jax.numpy as jnp
from jax import lax
from jax.experimental import pallas as pl
from jax.experimental.pallas import tpu as pltpu
from jax.experimental.pallas import tpu_sc as plsc

N_NODES = 10000
N_EDGES = 320000
NQ = 8
DIM = 256
N_GRAPHS = 128
OUT = 2

FW = 16
N_PAD = 10240
DUMMY = N_NODES
EB = 128
NC, NS = 2, 16
NW = NC * NS
K_PER_W = 80
E_PAD = EB * K_PER_W * NW
ROWS_PER_TILE = N_PAD // NS

BLK = 1024
NBLK = N_PAD // BLK

_HIGH = lax.Precision.HIGHEST


def _split_bf16(a):
    hi = a.astype(jnp.bfloat16)
    lo = (a - hi.astype(jnp.float32)).astype(jnp.bfloat16)
    return hi, lo


def _mm(a, b, dims=(((1,), (0,)), ((), ()))):
    return lax.dot_general(a, b, dims, preferred_element_type=jnp.float32)


def _dot3(a, b):
    ah, al = _split_bf16(a)
    bh, bl = _split_bf16(b)
    return _mm(ah, bh) + (_mm(al, bh) + _mm(ah, bl))


def _dot2(a, b_exact, dims=(((1,), (0,)), ((), ()))):
    ah, al = _split_bf16(a)
    bh = b_exact.astype(jnp.bfloat16)
    return _mm(ah, bh, dims) + _mm(al, bh, dims)


def _cnot_chain_matrix():
    i = np.arange(DIM)
    T = np.eye(DIM, dtype=np.float32)
    for q in range(NQ - 1):
        cb = (i >> (NQ - 1 - q)) & 1
        newi = np.where(cb == 1, i ^ (1 << (NQ - 2 - q)), i)
        Tq = np.zeros((DIM, DIM), np.float32)
        Tq[i, newi] = 1.0
        T = T @ Tq
    return T


_TCHAIN = _cnot_chain_matrix()
_G = (1.0 - 2.0 * ((np.arange(DIM)[:, None] >> (NQ - 1 - np.arange(NQ)[None, :])) & 1)).astype(np.float32)

NBUF = 8
K_FAST, K_SLOW = 120, 40


def _sc_body(gather, vals_hbm, row2_hbm, col2_hbm, zeros_hbm, out_hbm,
             ridx_v, cidx_v, vals_v, stage_v, acc_sh, *sems):
    cid = lax.axis_index("c")
    sid = lax.axis_index("s")
    wid = sid * NC + cid
    pltpu.sync_copy(zeros_hbm, acc_sh.at[pl.ds(sid * ROWS_PER_TILE, ROWS_PER_TILE)])
    plsc.subcore_barrier()

    if gather:
        def pipe(k, cbase):
            pltpu.sync_copy(col2_hbm.at[pl.ds(cbase, k)], cidx_v.at[pl.ds(0, k)])
            pltpu.sync_copy(row2_hbm.at[pl.ds(cbase, k)], ridx_v.at[pl.ds(0, k)])
            for b in range(NBUF):
                pltpu.async_copy(vals_hbm.at[ridx_v.at[b]], vals_v.at[b], sems[b])

            def step(m, carry):
                for b in range(NBUF):
                    j = m * NBUF + b
                    pltpu.make_async_copy(
                        vals_hbm.at[ridx_v.at[j]], vals_v.at[b], sems[b]).wait()
                    pltpu.sync_copy(vals_v.at[b], acc_sh.at[cidx_v.at[j]], add=True)

                    @pl.when(j + NBUF < k)
                    def _():
                        pltpu.async_copy(
                            vals_hbm.at[ridx_v.at[j + NBUF]], vals_v.at[b], sems[b])
                return carry

            lax.fori_loop(0, k // NBUF, step, 0)

        @pl.when(cid == 0)
        def _():
            pipe(K_FAST, sid * K_FAST)

        @pl.when(cid == 1)
        def _():
            pipe(K_SLOW, NS * K_FAST + sid * K_SLOW)
    else:
        pltpu.sync_copy(col2_hbm.at[pl.ds(wid * K_PER_W, K_PER_W)], cidx_v)
        pltpu.sync_copy(vals_hbm, vals_v.at[0])

        def step(j, carry):
            pltpu.sync_copy(vals_v.at[0], acc_sh.at[cidx_v.at[j]], add=True)
            return carry

        lax.fori_loop(0, K_PER_W, step, 0)
    plsc.subcore_barrier()
    pltpu.sync_copy(acc_sh.at[pl.ds(sid * ROWS_PER_TILE, ROWS_PER_TILE)], stage_v)
    pltpu.sync_copy(
        stage_v, out_hbm.at[pl.ds(cid * N_PAD + sid * ROWS_PER_TILE, ROWS_PER_TILE)])


@functools.lru_cache(maxsize=None)
def _make_sc(gather):
    mesh = plsc.VectorSubcoreMesh(
        core_axis_name="c", subcore_axis_name="s", num_cores=NC, num_subcores=NS)
    kmax = K_FAST if gather else K_PER_W
    scratch = [
        pltpu.VMEM((kmax, EB), jnp.int32),
        pltpu.VMEM((kmax, EB), jnp.int32),
        pltpu.VMEM((NBUF, EB, FW), jnp.float32),
        pltpu.VMEM((ROWS_PER_TILE, FW), jnp.float32),
        pltpu.VMEM_SHARED((N_PAD, FW), jnp.float32),
    ] + [pltpu.SemaphoreType.DMA] * NBUF
    return pl.kernel(
        functools.partial(_sc_body, gather),
        out_type=jax.ShapeDtypeStruct((NC * N_PAD, FW), jnp.float32),
        mesh=mesh,
        scratch_types=scratch,
        compiler_params=pltpu.CompilerParams(use_tc_tiling_on_sc=False),
        name="sc_edge_scatter" if gather else "sc_degree",
    )


def _sc_scatter(*args):
    return _make_sc(True)(*args)


def _sc_degree(*args):
    return _make_sc(False)(*args)


def _prologue_body(x_ref, deg_ref, xs_ref, dinv_ref):
    cnt = deg_ref[:N_PAD, 0:1] + deg_ref[N_PAD:, 0:1]
    dinv = lax.rsqrt(cnt + 1.0)
    xs8 = x_ref[...] * dinv
    ones = jnp.ones((N_PAD, 1), jnp.float32)
    zeros = jnp.zeros((N_PAD, FW - NQ - 1), jnp.float32)
    xs_ref[...] = jnp.concatenate([xs8, ones, zeros], axis=1)
    dinv_ref[...] = jnp.broadcast_to(dinv, (N_PAD, NQ))


def _prologue(x_pad, degacc):
    return pl.pallas_call(
        _prologue_body,
        out_shape=(
            jax.ShapeDtypeStruct((N_PAD, FW), jnp.float32),
            jax.ShapeDtypeStruct((N_PAD, NQ), jnp.float32),
        ),
    )(x_pad, degacc)


def _build_t_body(w0_ref, w1_ref, tc_ref, ta_ref, tb_ref):
    jb = lax.broadcasted_iota(jnp.int32, (DIM, DIM), 0)
    ib = lax.broadcasted_iota(jnp.int32, (DIM, DIM), 1)
    for w_ref, t_ref in ((w0_ref, ta_ref), (w1_ref, tb_ref)):
        T = tc_ref[...]
        for l in range(2):
            TR = jnp.ones((DIM, DIM), jnp.float32)
            for q in range(NQ):
                ang = w_ref[l, q] * 0.5
                c = jnp.cos(ang)
                s = jnp.sin(ang)
                jbq = (jb >> (NQ - 1 - q)) & 1
                ibq = (ib >> (NQ - 1 - q)) & 1
                TR = TR * jnp.where(jbq == ibq, c, jnp.where(jbq > ibq, -s, s))
            T = jnp.dot(T, TR, precision=_HIGH, preferred_element_type=jnp.float32)
            if l == 0:
                T = jnp.dot(T, tc_ref[...], precision=_HIGH,
                            preferred_element_type=jnp.float32)
        t_ref[...] = T


def _build_t(w0, w1, tchain):
    return pl.pallas_call(
        _build_t_body,
        in_specs=[
            pl.BlockSpec(memory_space=pltpu.SMEM),
            pl.BlockSpec(memory_space=pltpu.SMEM),
            pl.BlockSpec((DIM, DIM), lambda: (0, 0)),
        ],
        out_shape=(
            jax.ShapeDtypeStruct((DIM, DIM), jnp.float32),
            jax.ShapeDtypeStruct((DIM, DIM), jnp.float32),
        ),
    )(w0, w1, tchain)


def _dense_body(first_layer, acca_ref, accb_ref, xs_ref, dinv_ref, t_ref, g_ref, out_ref):
    dinv8 = dinv_ref[...]
    feats = dinv8 * (acca_ref[:, :NQ] + accb_ref[:, :NQ] + xs_ref[:, :NQ])
    phi = feats * 0.5 + (np.pi / 4.0)
    cv = jnp.cos(phi)
    sv = jnp.sin(phi)
    ib = lax.broadcasted_iota(jnp.int32, (1, DIM), 1)
    S = jnp.ones((BLK, DIM), jnp.float32)
    for q in range(NQ):
        bit = (ib >> (NQ - 1 - q)) & 1
        S = S * jnp.where(bit == 1, sv[:, q:q + 1], cv[:, q:q + 1])
    F = _dot3(S, t_ref[...])
    P = F * F
    z = _dot2(P, g_ref[...])
    h = jnp.where(z >= 0.0, z, 0.2 * z)
    if first_layer:
        ones = jnp.ones((BLK, 1), jnp.float32)
        zeros = jnp.zeros((BLK, FW - NQ - 1), jnp.float32)
        out_ref[...] = jnp.concatenate([dinv8 * h, ones, zeros], axis=1)
    else:
        out_ref[...] = h


def _dense(first_layer, acc, xs16, dinv8, T, G):
    width = FW if first_layer else NQ
    return pl.pallas_call(
        functools.partial(_dense_body, first_layer),
        grid=(NBLK,),
        in_specs=[
            pl.BlockSpec((BLK, FW), lambda i: (i, 0)),
            pl.BlockSpec((BLK, FW), lambda i: (i + NBLK, 0)),
            pl.BlockSpec((BLK, FW), lambda i: (i, 0)),
            pl.BlockSpec((BLK, NQ), lambda i: (i, 0)),
            pl.BlockSpec((DIM, DIM), lambda i: (0, 0)),
            pl.BlockSpec((DIM, NQ), lambda i: (0, 0)),
        ],
        out_specs=pl.BlockSpec((BLK, width), lambda i: (i, 0)),
        out_shape=jax.ShapeDtypeStruct((N_PAD, width), jnp.float32),
    )(acc, acc, xs16, dinv8, T, G)


def _pool_body(h_ref, batch_ref, w_ref, b_ref, out_ref, sums, cnts):
    i = pl.program_id(0)

    @pl.when(i == 0)
    def _():
        sums[...] = jnp.zeros((N_GRAPHS, NQ), jnp.float32)
        cnts[...] = jnp.zeros((N_GRAPHS, 1), jnp.float32)

    onehot = (batch_ref[...] == lax.broadcasted_iota(jnp.int32, (1, N_GRAPHS), 1)
              ).astype(jnp.bfloat16)
    hh, hl = _split_bf16(h_ref[...])
    tdims = (((0,), (0,)), ((), ()))
    sums[...] += _mm(onehot, hh, tdims) + _mm(onehot, hl, tdims)
    cnts[...] += _mm(onehot, jnp.ones((BLK, 1), jnp.bfloat16), tdims)

    @pl.when(i == NBLK - 1)
    def _():
        pooled = sums[...] / jnp.maximum(cnts[...], 1.0)
        out_ref[...] = lax.dot_general(pooled, w_ref[...], (((1,), (1,)), ((), ())),
                                       precision=_HIGH,
                                       preferred_element_type=jnp.float32) + b_ref[...]


def _pool(h2, batch2, W, b2):
    return pl.pallas_call(
        _pool_body,
        grid=(NBLK,),
        in_specs=[
            pl.BlockSpec((BLK, NQ), lambda i: (i, 0)),
            pl.BlockSpec((BLK, 1), lambda i: (i, 0)),
            pl.BlockSpec((OUT, NQ), lambda i: (0, 0)),
            pl.BlockSpec((1, OUT), lambda i: (0, 0)),
        ],
        out_specs=pl.BlockSpec((N_GRAPHS, OUT), lambda i: (0, 0)),
        out_shape=jax.ShapeDtypeStruct((N_GRAPHS, OUT), jnp.float32),
        scratch_shapes=[
            pltpu.VMEM((N_GRAPHS, NQ), jnp.float32),
            pltpu.VMEM((N_GRAPHS, 1), jnp.float32),
        ],
    )(h2, batch2, W, b2)


def kernel(x, edge_index, batch, q_weights_0, q_weights_1, W, b):
    row = edge_index[0].astype(jnp.int32)
    col = edge_index[1].astype(jnp.int32)
    pad = jnp.full((E_PAD - N_EDGES,), DUMMY, jnp.int32)
    row2 = jnp.concatenate([row, pad]).reshape(NW * K_PER_W, EB)
    col2 = jnp.concatenate([col, pad]).reshape(NW * K_PER_W, EB)
    zeros_tile = jnp.zeros((ROWS_PER_TILE, FW), jnp.float32)
    ones_eb = jnp.ones((EB, FW), jnp.float32)
    x_pad = jnp.concatenate(
        [x, jnp.zeros((N_PAD - N_NODES, NQ), jnp.float32)], axis=0)
    batch2 = jnp.concatenate(
        [batch.astype(jnp.int32), jnp.full((N_PAD - N_NODES,), -1, jnp.int32)]
    ).reshape(N_PAD, 1)

    degacc = _sc_degree(ones_eb, row2, col2, zeros_tile)
    xs16, dinv8 = _prologue(x_pad, degacc)
    Ta, Tb = _build_t(q_weights_0, q_weights_1, jnp.asarray(_TCHAIN))

    acc1 = _sc_scatter(xs16, row2, col2, zeros_tile)
    xs16_2 = _dense(True, acc1, xs16, dinv8, Ta, jnp.asarray(_G))
    acc2 = _sc_scatter(xs16_2, row2, col2, zeros_tile)
    h2 = _dense(False, acc2, xs16_2, dinv8, Tb, jnp.asarray(_G))
    return _pool(h2, batch2, W, b.reshape(1, OUT))
